# Initial kernel scaffold; baseline (speedup 1.0000x reference)
import jax
import jax.numpy as jnp
from jax.experimental import pallas as pl


def kernel(features, W1, b1, W2, b2):
    raise NotImplementedError("write your pallas kernel here")


# speedup vs baseline: 13.0817x; 13.0817x over previous
import functools

import jax
import jax.numpy as jnp
from jax.experimental import pallas as pl
from jax.experimental.pallas import tpu as pltpu

K_TOPK = 31


def _emb_kernel(f_ref, w1_ref, b1_ref, w2_ref, b2_ref, out_ref):
    f = f_ref[...]
    h = jax.lax.dot_general(f, w1_ref[...], (((1,), (1,)), ((), ())),
                            preferred_element_type=jnp.float32)
    h = h + b1_ref[0:1, :]
    h = jnp.maximum(h, 0.0)
    h = jax.lax.dot_general(h, w2_ref[...], (((1,), (1,)), ((), ())),
                            preferred_element_type=jnp.float32)
    h = h + b2_ref[0:1, :]
    norm = jnp.sqrt(jnp.sum(h * h, axis=1, keepdims=True))
    out_ref[...] = h / jnp.maximum(norm, 1e-12)


def _topk_kernel(emb_row_ref, emb_all_ref, out_ref, sim_ref, work_ref):
    sim = jax.lax.dot_general(
        emb_row_ref[...], emb_all_ref[...], (((1,), (1,)), ((), ())),
        preferred_element_type=jnp.float32)
    sim_ref[...] = sim
    work_ref[...] = sim

    def body(i, _):
        m = jnp.max(work_ref[...], axis=1, keepdims=True)
        work_ref[...] = jnp.where(work_ref[...] >= m, -3.0, work_ref[...])
        return 0

    jax.lax.fori_loop(0, K_TOPK - 1, body, 0)
    thresh = jnp.max(work_ref[...], axis=1, keepdims=True)
    s = sim_ref[...]
    out_ref[...] = jnp.where(s >= thresh, jnp.maximum(s, 0.0), 0.0)


def kernel(features, W1, b1, W2, b2):
    n, d = features.shape
    eb = n // 5 if n % 5 == 0 else n
    emb = pl.pallas_call(
        _emb_kernel,
        grid=(n // eb,),
        in_specs=[
            pl.BlockSpec((eb, d), lambda i: (i, 0)),
            pl.BlockSpec((d, d), lambda i: (0, 0)),
            pl.BlockSpec((8, d), lambda i: (0, 0)),
            pl.BlockSpec((d, d), lambda i: (0, 0)),
            pl.BlockSpec((8, d), lambda i: (0, 0)),
        ],
        out_specs=pl.BlockSpec((eb, d), lambda i: (i, 0)),
        out_shape=jax.ShapeDtypeStruct((n, d), jnp.float32),
    )(features, W1, jnp.broadcast_to(b1, (8, d)), W2,
      jnp.broadcast_to(b2, (8, d)))

    r = 200 if n % 200 == 0 else n
    out = pl.pallas_call(
        _topk_kernel,
        grid=(n // r,),
        in_specs=[
            pl.BlockSpec((r, d), lambda i: (i, 0)),
            pl.BlockSpec((n, d), lambda i: (0, 0)),
        ],
        out_specs=pl.BlockSpec((r, n), lambda i: (i, 0)),
        out_shape=jax.ShapeDtypeStruct((n, n), jnp.float32),
        scratch_shapes=[
            pltpu.VMEM((r, n), jnp.float32),
            pltpu.VMEM((r, n), jnp.float32),
        ],
        compiler_params=pltpu.CompilerParams(
            dimension_semantics=("arbitrary",),
        ),
    )(emb, emb)
    return out


# speedup vs baseline: 17.8636x; 1.3655x over previous
import functools

import jax
import jax.numpy as jnp
from jax.experimental import pallas as pl
from jax.experimental.pallas import tpu as pltpu

K_TOPK = 31
BSEARCH_ITERS = 20
NEG = -3.0


def _emb_kernel(f_ref, w1_ref, b1_ref, w2_ref, b2_ref, out_ref):
    f = f_ref[...]
    h = jax.lax.dot_general(f, w1_ref[...], (((1,), (1,)), ((), ())),
                            preferred_element_type=jnp.float32)
    h = h + b1_ref[0:1, :]
    h = jnp.maximum(h, 0.0)
    h = jax.lax.dot_general(h, w2_ref[...], (((1,), (1,)), ((), ())),
                            preferred_element_type=jnp.float32)
    h = h + b2_ref[0:1, :]
    norm = jnp.sqrt(jnp.sum(h * h, axis=1, keepdims=True))
    out_ref[...] = h / jnp.maximum(norm, 1e-12)


def _topk_kernel(n, emb_row_ref, emb_all_ref, out_ref, sim_ref):
    np_ = sim_ref.shape[1]
    sim_ref[:, :n] = jax.lax.dot_general(
        emb_row_ref[...], emb_all_ref[...], (((1,), (1,)), ((), ())),
        preferred_element_type=jnp.float32)
    if np_ > n:
        sim_ref[:, n:] = jnp.full((sim_ref.shape[0], np_ - n), NEG,
                                  jnp.float32)
    s = sim_ref[...]

    g = s[:, 0:128]
    for c in range(1, np_ // 128):
        g = jnp.maximum(g, s[:, c * 128:(c + 1) * 128])
    hi = jnp.max(g, axis=1, keepdims=True)
    for _ in range(K_TOPK - 1):
        m = jnp.max(g, axis=1, keepdims=True)
        g = jnp.where(g >= m, NEG, g)
    lo = jnp.max(g, axis=1, keepdims=True)

    def body(i, carry):
        lo, hi = carry
        mid = 0.5 * (lo + hi)
        cnt = jnp.sum(jnp.where(s >= mid, 1.0, 0.0), axis=1, keepdims=True)
        ok = cnt >= K_TOPK
        return jnp.where(ok, mid, lo), jnp.where(ok, hi, mid)

    lo, hi = jax.lax.fori_loop(0, BSEARCH_ITERS, body, (lo, hi))
    sv = s[:, :n]
    out_ref[...] = jnp.where(sv >= lo, jnp.maximum(sv, 0.0), 0.0)


def kernel(features, W1, b1, W2, b2):
    n, d = features.shape
    eb = n // 5 if n % 5 == 0 else n
    emb = pl.pallas_call(
        _emb_kernel,
        grid=(n // eb,),
        in_specs=[
            pl.BlockSpec((eb, d), lambda i: (i, 0)),
            pl.BlockSpec((d, d), lambda i: (0, 0)),
            pl.BlockSpec((8, d), lambda i: (0, 0)),
            pl.BlockSpec((d, d), lambda i: (0, 0)),
            pl.BlockSpec((8, d), lambda i: (0, 0)),
        ],
        out_specs=pl.BlockSpec((eb, d), lambda i: (i, 0)),
        out_shape=jax.ShapeDtypeStruct((n, d), jnp.float32),
    )(features, W1, jnp.broadcast_to(b1, (8, d)), W2,
      jnp.broadcast_to(b2, (8, d)))

    r = 200 if n % 200 == 0 else n
    npad = ((n + 127) // 128) * 128
    out = pl.pallas_call(
        functools.partial(_topk_kernel, n),
        grid=(n // r,),
        in_specs=[
            pl.BlockSpec((r, d), lambda i: (i, 0)),
            pl.BlockSpec((n, d), lambda i: (0, 0)),
        ],
        out_specs=pl.BlockSpec((r, n), lambda i: (i, 0)),
        out_shape=jax.ShapeDtypeStruct((n, n), jnp.float32),
        scratch_shapes=[
            pltpu.VMEM((r, npad), jnp.float32),
        ],
        compiler_params=pltpu.CompilerParams(
            dimension_semantics=("arbitrary",),
        ),
    )(emb, emb)
    return out


# speedup vs baseline: 32.9171x; 1.8427x over previous
import functools

import jax
import jax.numpy as jnp
from jax.experimental import pallas as pl
from jax.experimental.pallas import tpu as pltpu

K_TOPK = 31
REPAIR_ITERS = 22
NEG = -3.0


def _emb_kernel(f_ref, w1_ref, b1_ref, w2_ref, b2_ref, out_ref):
    f = f_ref[...]
    h = jax.lax.dot_general(f, w1_ref[...], (((1,), (1,)), ((), ())),
                            preferred_element_type=jnp.float32)
    h = h + b1_ref[0:1, :]
    h = jnp.maximum(h, 0.0)
    h = jax.lax.dot_general(h, w2_ref[...], (((1,), (1,)), ((), ())),
                            preferred_element_type=jnp.float32)
    h = h + b2_ref[0:1, :]
    norm = jnp.sqrt(jnp.sum(h * h, axis=1, keepdims=True))
    out_ref[...] = h / jnp.maximum(norm, 1e-12)


def _topk_kernel(n, emb_row_ref, emb_all_ref, out_ref, c_ref):
    r = out_ref.shape[0]
    eb = emb_row_ref[...]
    g1 = jnp.full((r, 128), NEG, jnp.float32)
    g2 = g1
    g3 = g1
    g4 = g1
    nch = (n + 127) // 128
    for c in range(nch):
        lo_c = c * 128
        w = min(128, n - lo_c)
        blk = emb_all_ref[lo_c:lo_c + w, :]
        v = jax.lax.dot_general(eb, blk, (((1,), (1,)), ((), ())),
                                preferred_element_type=jnp.float32)
        out_ref[:, lo_c:lo_c + w] = v
        if w < 128:
            v = jnp.concatenate(
                [v, jnp.full((r, 128 - w), NEG, jnp.float32)], axis=1)
        a = jnp.maximum(g1, v)
        b = jnp.minimum(g1, v)
        g1 = a
        a = jnp.maximum(g2, b)
        b = jnp.minimum(g2, b)
        g2 = a
        a = jnp.maximum(g3, b)
        b = jnp.minimum(g3, b)
        g3 = a
        g4 = jnp.maximum(g4, b)

    m1 = jnp.max(g1, axis=1, keepdims=True)
    c_ref[:, 0:128] = g1
    c_ref[:, 128:256] = g2
    c_ref[:, 256:384] = g3
    c_ref[:, 384:512] = g4

    def sbody(i, _):
        m = jnp.max(c_ref[...], axis=1, keepdims=True)
        c_ref[...] = jnp.where(c_ref[...] >= m, NEG, c_ref[...])
        return 0

    jax.lax.fori_loop(0, K_TOPK - 1, sbody, 0)
    thr = jnp.max(c_ref[...], axis=1, keepdims=True)

    sv = out_ref[:, :n]
    cnt = jnp.sum(jnp.where(sv >= thr, 1.0, 0.0), axis=1, keepdims=True)
    bad = jnp.any(cnt > K_TOPK + 0.5)

    @pl.when(jnp.logical_not(bad))
    def _():
        s = out_ref[:, :n]
        out_ref[:, :n] = jnp.where(s >= thr, jnp.maximum(s, 0.0), 0.0)

    @pl.when(bad)
    def _():
        def rbody(i, carry):
            lo, hi = carry
            mid = 0.5 * (lo + hi)
            c2 = jnp.sum(jnp.where(out_ref[:, :n] >= mid, 1.0, 0.0),
                         axis=1, keepdims=True)
            ok = c2 >= K_TOPK
            return jnp.where(ok, mid, lo), jnp.where(ok, hi, mid)

        lo2, _hi2 = jax.lax.fori_loop(0, REPAIR_ITERS, rbody, (thr, m1))
        s = out_ref[:, :n]
        out_ref[:, :n] = jnp.where(s >= lo2, jnp.maximum(s, 0.0), 0.0)


def kernel(features, W1, b1, W2, b2):
    n, d = features.shape
    eb = n // 5 if n % 5 == 0 else n
    emb = pl.pallas_call(
        _emb_kernel,
        grid=(n // eb,),
        in_specs=[
            pl.BlockSpec((eb, d), lambda i: (i, 0)),
            pl.BlockSpec((d, d), lambda i: (0, 0)),
            pl.BlockSpec((8, d), lambda i: (0, 0)),
            pl.BlockSpec((d, d), lambda i: (0, 0)),
            pl.BlockSpec((8, d), lambda i: (0, 0)),
        ],
        out_specs=pl.BlockSpec((eb, d), lambda i: (i, 0)),
        out_shape=jax.ShapeDtypeStruct((n, d), jnp.float32),
    )(features, W1, jnp.broadcast_to(b1, (8, d)), W2,
      jnp.broadcast_to(b2, (8, d)))

    r = 200 if n % 200 == 0 else n
    out = pl.pallas_call(
        functools.partial(_topk_kernel, n),
        grid=(n // r,),
        in_specs=[
            pl.BlockSpec((r, d), lambda i: (i, 0)),
            pl.BlockSpec((n, d), lambda i: (0, 0)),
        ],
        out_specs=pl.BlockSpec((r, n), lambda i: (i, 0)),
        out_shape=jax.ShapeDtypeStruct((n, n), jnp.float32),
        scratch_shapes=[
            pltpu.VMEM((r, 512), jnp.float32),
        ],
        compiler_params=pltpu.CompilerParams(
            dimension_semantics=("arbitrary",),
        ),
    )(emb, emb)
    return out
